# baseline (speedup 1.0000x reference)
import functools

import jax
import jax.numpy as jnp
from jax import lax
from jax.experimental import pallas as pl
from jax.experimental.pallas import tpu as pltpu
from jax.experimental.pallas import tpu_sc as plsc

N_POS = 8192
D_MODEL = 1024
N_CORES = 2
N_SUBCORES = 16
N_WORKERS = N_CORES * N_SUBCORES

TOTAL_WORDS = N_POS * D_MODEL
WORDS_PER_W = TOTAL_WORDS // N_WORKERS

CHUNK = 65528
_full = WORDS_PER_W // CHUNK
_tail = WORDS_PER_W - _full * CHUNK
CHUNK_SIZES = [CHUNK] * _full + ([_tail] if _tail else [])
CHUNK_OFFS = [sum(CHUNK_SIZES[:i]) for i in range(len(CHUNK_SIZES))]
N_CHUNKS = len(CHUNK_SIZES)


@functools.partial(
    pl.kernel,
    mesh=plsc.VectorSubcoreMesh(core_axis_name="c", subcore_axis_name="s"),
    out_type=jax.ShapeDtypeStruct((TOTAL_WORDS,), jnp.float32),
    scratch_types=[
        pltpu.VMEM((CHUNK,), jnp.float32),
        pltpu.VMEM((CHUNK,), jnp.float32),
        pltpu.SemaphoreType.DMA,
        pltpu.SemaphoreType.DMA,
        pltpu.SemaphoreType.DMA,
        pltpu.SemaphoreType.DMA,
    ],
)
def _sc_copy(table_hbm, out_hbm, buf0, buf1, rsem0, rsem1, wsem0, wsem1):
    wid = lax.axis_index("s") * N_CORES + lax.axis_index("c")
    base = wid * WORDS_PER_W
    bufs = (buf0, buf1)
    rsems = (rsem0, rsem1)
    wsems = (wsem0, wsem1)

    def rd(i, b):
        n = CHUNK_SIZES[i]
        return pltpu.make_async_copy(
            table_hbm.at[pl.ds(base + CHUNK_OFFS[i], n)],
            bufs[b].at[pl.ds(0, n)], rsems[b])

    def wr(i, b):
        n = CHUNK_SIZES[i]
        return pltpu.make_async_copy(
            bufs[b].at[pl.ds(0, n)],
            out_hbm.at[pl.ds(base + CHUNK_OFFS[i], n)], wsems[b])

    rd(0, 0).start()
    rd(1, 1).start()
    for i in range(N_CHUNKS):
        b = i % 2
        rd(i, b).wait()
        wr(i, b).start()
        if i + 2 < N_CHUNKS:
            wr(i, b).wait()
            rd(i + 2, b).start()
    wr(N_CHUNKS - 2, N_CHUNKS % 2).wait()
    wr(N_CHUNKS - 1, (N_CHUNKS - 1) % 2).wait()


def kernel(x, positional_embedding):
    del x
    flat = positional_embedding.reshape(TOTAL_WORDS)
    return _sc_copy(flat).reshape(N_POS, D_MODEL)


# speedup vs baseline: 2.5093x; 2.5093x over previous
import functools

import jax
import jax.numpy as jnp
from jax import lax
from jax.experimental import pallas as pl
from jax.experimental.pallas import tpu as pltpu
from jax.experimental.pallas import tpu_sc as plsc

N_POS = 8192
D_MODEL = 1024
N_CORES = 2
N_SUBCORES = 16
N_WORKERS = N_CORES * N_SUBCORES
ROWS_PER_W = N_POS // N_WORKERS

CHUNK = 56
_full = ROWS_PER_W // CHUNK
_tail = ROWS_PER_W - _full * CHUNK
CHUNK_SIZES = [CHUNK] * _full + ([_tail] if _tail else [])
CHUNK_OFFS = [sum(CHUNK_SIZES[:i]) for i in range(len(CHUNK_SIZES))]
N_CHUNKS = len(CHUNK_SIZES)


@functools.partial(
    pl.kernel,
    mesh=plsc.VectorSubcoreMesh(core_axis_name="c", subcore_axis_name="s"),
    out_type=jax.ShapeDtypeStruct((N_POS, D_MODEL), jnp.float32),
    scratch_types=[
        pltpu.VMEM((CHUNK, D_MODEL), jnp.float32),
        pltpu.VMEM((CHUNK, D_MODEL), jnp.float32),
        pltpu.SemaphoreType.DMA,
        pltpu.SemaphoreType.DMA,
        pltpu.SemaphoreType.DMA,
        pltpu.SemaphoreType.DMA,
    ],
)
def _sc_copy(table_hbm, out_hbm, buf0, buf1, rsem0, rsem1, wsem0, wsem1):
    wid = lax.axis_index("s") * N_CORES + lax.axis_index("c")
    base = wid * ROWS_PER_W
    bufs = (buf0, buf1)
    rsems = (rsem0, rsem1)
    wsems = (wsem0, wsem1)

    def rd(i, b):
        n = CHUNK_SIZES[i]
        return pltpu.make_async_copy(
            table_hbm.at[pl.ds(base + CHUNK_OFFS[i], n)],
            bufs[b].at[pl.ds(0, n)], rsems[b])

    def wr(i, b):
        n = CHUNK_SIZES[i]
        return pltpu.make_async_copy(
            bufs[b].at[pl.ds(0, n)],
            out_hbm.at[pl.ds(base + CHUNK_OFFS[i], n)], wsems[b])

    rd(0, 0).start()
    rd(1, 1).start()
    for i in range(N_CHUNKS):
        b = i % 2
        rd(i, b).wait()
        wr(i, b).start()
        if i + 2 < N_CHUNKS:
            wr(i, b).wait()
            rd(i + 2, b).start()
    wr(N_CHUNKS - 2, N_CHUNKS % 2).wait()
    wr(N_CHUNKS - 1, (N_CHUNKS - 1) % 2).wait()


def kernel(x, positional_embedding):
    del x
    return _sc_copy(positional_embedding)
